# baseline (speedup 1.0000x reference)
import functools
import jax
import jax.numpy as jnp
from jax import lax
from jax.experimental import pallas as pl
from jax.experimental.pallas import tpu as pltpu
from jax.experimental.pallas import tpu_sc as plsc

NU = 10000
NI = 10000
NT = 2000
D = 128
NC = 2
NS = 16
L = 16

W = 80
EPT = 10000
NWIN = EPT // W
CH = 80
NCH_T = NT // CH
NCH_A = NU // CH
W2 = 128
NWIN2 = 80
EPT2 = W2 * NWIN2

_mesh = plsc.VectorSubcoreMesh(
    core_axis_name="c", subcore_axis_name="s", num_cores=NC, num_subcores=NS)


def _zero16():
    return jnp.zeros((L,), jnp.float32)


def _zero_buf(buf, nrows):
    z = _zero16()

    def zr(r, _):
        for d in range(D // L):
            buf[r, pl.ds(d * L, L)] = z
        return 0

    lax.fori_loop(0, nrows, zr, 0)


def _edge_windows(tidx, cr_hbm, va_hbm, src, acc_sp, dummy_hbm, bufs,
                  wsz=W, nwin=NWIN):
    cr = bufs[0:3]
    va = bufs[3:6]
    rb = bufs[6:9]
    g = bufs[9:12]
    gsem = bufs[12:15]
    ssem = bufs[15:18]
    isem = bufs[18:21]
    base = tidx * nwin

    def idx_start(w, q):
        pltpu.async_copy(cr_hbm.at[base + w], cr[q], isem[q])
        pltpu.async_copy(va_hbm.at[base + w], va[q], isem[q])

    def idx_wait(w, q):
        pltpu.make_async_copy(cr_hbm.at[base + w], cr[q], isem[q]).wait()
        pltpu.make_async_copy(va_hbm.at[base + w], va[q], isem[q]).wait()

    def buf_drain(sem, q):
        pltpu.make_async_copy(dummy_hbm.at[pl.ds(0, wsz)], g[q], sem).wait()

    def scale(p):
        def sgroup(gi, _):
            sl16 = pl.ds(gi * L, L)
            rb[p][sl16] = cr[p][1, sl16]
            v16 = va[p][0, sl16]
            for j in range(L):
                vv = jnp.full((L,), v16[j], jnp.float32)
                e = gi * L + j
                for d in range(D // L):
                    sl = pl.ds(d * L, L)
                    g[p][e, sl] = g[p][e, sl] * vv
            return 0

        lax.fori_loop(0, wsz // L, sgroup, 0)

    pltpu.sync_copy(cr_hbm.at[base], cr[0])
    pltpu.sync_copy(va_hbm.at[base], va[0])
    pltpu.async_copy(src.at[cr[0].at[0]], g[0], gsem[0])
    idx_start(1, 1)
    idx_start(2, 2)

    def k_iter(k, _):
        for p in range(3):
            w = 3 * k + p
            pn = (p + 1) % 3

            @pl.when(w < nwin)
            def _():
                buf_drain(gsem[p], p)

                @pl.when(w + 1 < nwin)
                def _():
                    @pl.when(w >= 2)
                    def _():
                        buf_drain(ssem[pn], pn)
                    idx_wait(w + 1, pn)
                    pltpu.async_copy(src.at[cr[pn].at[0]], g[pn], gsem[pn])

                scale(p)
                pltpu.async_copy(g[p], acc_sp.at[rb[p]], ssem[p], add=True)

                @pl.when(w + 3 < nwin)
                def _():
                    idx_start(w + 3, p)

        return 0

    lax.fori_loop(0, (nwin + 2) // 3, k_iter, 0)
    for q in range(3):
        buf_drain(ssem[q], q)


def _phase_a_body(tag_hbm, u_cr, u_va, i_cr, i_va,
                  ui_out, tag_sp, acc_sp, *bufs):
    c = lax.axis_index("c")
    s = lax.axis_index("s")
    g0, g1 = bufs[9], bufs[10]

    for j in range((NCH_T + NS - 1) // NS):
        k = s + j * NS

        @pl.when(k < NCH_T)
        def _():
            pltpu.sync_copy(tag_hbm.at[pl.ds(k * CH, CH)], g1)
            pltpu.sync_copy(g1, tag_sp.at[pl.ds(k * CH, CH)])

    _zero_buf(g0, W)
    for j in range((NCH_A + NS - 1) // NS):
        k = s + j * NS

        @pl.when(k < NCH_A)
        def _():
            pltpu.sync_copy(g0, acc_sp.at[pl.ds(k * CH, CH)])

    plsc.subcore_barrier()

    @pl.when(c == 0)
    def _():
        _edge_windows(s, u_cr, u_va, tag_sp, acc_sp, ui_out, bufs)

    @pl.when(c == 1)
    def _():
        _edge_windows(s, i_cr, i_va, tag_sp, acc_sp, ui_out, bufs)

    plsc.subcore_barrier()
    for j in range((NCH_A + NS - 1) // NS):
        k = s + j * NS

        @pl.when(k < NCH_A)
        def _():
            pltpu.sync_copy(acc_sp.at[pl.ds(k * CH, CH)],
                            ui_out.at[pl.ds(c * NU + k * CH, CH)])


def _make_pipe_scratch(wsz):
    return (
        [pltpu.VMEM((2, wsz), jnp.int32)] * 3
        + [pltpu.VMEM((1, wsz), jnp.float32)] * 3
        + [pltpu.VMEM((wsz,), jnp.int32)] * 3
        + [pltpu.VMEM((wsz, D), jnp.float32)] * 3
        + [pltpu.SemaphoreType.DMA] * 9
    )


_pipe_scratch = _make_pipe_scratch(W)

_phase_a = pl.kernel(
    _phase_a_body,
    out_type=jax.ShapeDtypeStruct((NU + NI, D), jnp.float32),
    mesh=_mesh,
    scratch_types=[
        pltpu.VMEM_SHARED((NT, D), jnp.float32),
        pltpu.VMEM_SHARED((NU, D), jnp.float32),
    ] + _pipe_scratch,
)


def _phase_b_body(ui_raw, t_cr, t_va, tp0, tp1, acc_sp, *bufs):
    c = lax.axis_index("c")
    s = lax.axis_index("s")
    g0 = bufs[9]

    _zero_buf(g0, CH)
    for j in range((NCH_T + NS - 1) // NS):
        k = s + j * NS

        @pl.when(k < NCH_T)
        def _():
            pltpu.sync_copy(g0.at[pl.ds(0, CH)], acc_sp.at[pl.ds(k * CH, CH)])

    plsc.subcore_barrier()

    _edge_windows(c * NS + s, t_cr, t_va, ui_raw, acc_sp, ui_raw, bufs,
                  wsz=W2, nwin=NWIN2)

    plsc.subcore_barrier()

    for j in range((NCH_T + NS - 1) // NS):
        k = s + j * NS

        @pl.when((k < NCH_T) & (c == 0))
        def _():
            pltpu.sync_copy(acc_sp.at[pl.ds(k * CH, CH)],
                            tp0.at[pl.ds(k * CH, CH)])

        @pl.when((k < NCH_T) & (c == 1))
        def _():
            pltpu.sync_copy(acc_sp.at[pl.ds(k * CH, CH)],
                            tp1.at[pl.ds(k * CH, CH)])


_phase_b = pl.kernel(
    _phase_b_body,
    out_type=(jax.ShapeDtypeStruct((NT, D), jnp.float32),
              jax.ShapeDtypeStruct((NT, D), jnp.float32)),
    mesh=_mesh,
    scratch_types=[
        pltpu.VMEM_SHARED((NT, D), jnp.float32),
    ] + _make_pipe_scratch(W2),
)


def _tc_norm(x):
    n = jnp.sqrt(jnp.sum(x * x, axis=1, keepdims=True))
    return x / jnp.maximum(n, 1e-12)


def _tc_ui_body(base, x1, x2, x3, o):
    o[...] = (base[...] + _tc_norm(x1[...]) + _tc_norm(x2[...]) / 2.0
              + _tc_norm(x3[...]) / 3.0)


def _tc_tag_body(base, a0, b0, a1, b1, a2, b2, o):
    o[...] = (base[...] + _tc_norm(a0[...] + b0[...])
              + _tc_norm(a1[...] + b1[...]) / 2.0
              + _tc_norm(a2[...] + b2[...]) / 3.0)


def _tc_tagnorm_body(a, b, o):
    o[...] = _tc_norm(a[...] + b[...])


_tc_tagnorm = pl.pallas_call(
    _tc_tagnorm_body,
    out_shape=jax.ShapeDtypeStruct((NT, D), jnp.float32),
    grid=(1,),
    in_specs=[pl.BlockSpec((NT, D), lambda i: (0, 0))] * 2,
    out_specs=pl.BlockSpec((NT, D), lambda i: (0, 0)),
)


_UI_BLK = 1000
_tc_ui = pl.pallas_call(
    _tc_ui_body,
    out_shape=jax.ShapeDtypeStruct((NU + NI, D), jnp.float32),
    grid=((NU + NI) // _UI_BLK,),
    in_specs=[pl.BlockSpec((_UI_BLK, D), lambda i: (i, 0))] * 4,
    out_specs=pl.BlockSpec((_UI_BLK, D), lambda i: (i, 0)),
)

_tc_tag = pl.pallas_call(
    _tc_tag_body,
    out_shape=jax.ShapeDtypeStruct((NT, D), jnp.float32),
    grid=(1,),
    in_specs=[pl.BlockSpec((NT, D), lambda i: (0, 0))] * 7,
    out_specs=pl.BlockSpec((NT, D), lambda i: (0, 0)),
)


def kernel(user_emb, item_emb, tag_emb,
           u_rows, u_cols, u_vals,
           i_rows, i_cols, i_vals,
           t_rows, t_cols, t_vals):
    def pack(cols, rows, vals):
        cr = jnp.stack([cols.reshape(-1, W), rows.reshape(-1, W)], axis=1)
        return cr, vals.reshape(-1, 1, W)

    def pack2(cols, rows, vals):
        cr = jnp.stack([cols.reshape(-1, W2), rows.reshape(-1, W2)], axis=1)
        return cr, vals.reshape(-1, 1, W2)

    u_cr, u_va = pack(u_cols, u_rows, u_vals)
    i_cr, i_va = pack(i_cols, i_rows, i_vals)
    npad = NC * NS * EPT2 - t_cols.shape[0]
    zi = jnp.zeros((npad,), t_cols.dtype)
    t_cr, t_va = pack2(jnp.concatenate([t_cols, zi]),
                       jnp.concatenate([t_rows, zi]),
                       jnp.concatenate([t_vals, jnp.zeros((npad,),
                                                          t_vals.dtype)]))

    ui1 = _phase_a(tag_emb, u_cr, u_va, i_cr, i_va)
    tp0_0, tp1_0 = _phase_b(ui1, t_cr, t_va)
    tag1 = _tc_tagnorm(tp0_0, tp1_0)
    ui2 = _phase_a(tag1, u_cr, u_va, i_cr, i_va)
    tp0_1, tp1_1 = _phase_b(ui2, t_cr, t_va)
    tag2 = _tc_tagnorm(tp0_1, tp1_1)
    ui3 = _phase_a(tag2, u_cr, u_va, i_cr, i_va)
    tp0_2, tp1_2 = _phase_b(ui3, t_cr, t_va)

    base_ui = jnp.concatenate([user_emb, item_emb], axis=0)
    out_ui = _tc_ui(base_ui, ui1, ui2, ui3)
    out_t = _tc_tag(tag_emb, tp0_0, tp1_0, tp0_1, tp1_1, tp0_2, tp1_2)
    return (out_ui[:NU], out_ui[NU:], out_t)


# speedup vs baseline: 2.5049x; 2.5049x over previous
import functools
import jax
import jax.numpy as jnp
from jax import lax
from jax.experimental import pallas as pl
from jax.experimental.pallas import tpu as pltpu
from jax.experimental.pallas import tpu_sc as plsc

NU = 10000
NI = 10000
NT = 2000
D = 128
NC = 2
NS = 16
L = 16

W = 80
EPT = 10000
NWIN = EPT // W
CH = 80
NCH_T = NT // CH
NCH_A = NU // CH
W2 = 128
NWIN2 = 80
EPT2 = W2 * NWIN2

_mesh = plsc.VectorSubcoreMesh(
    core_axis_name="c", subcore_axis_name="s", num_cores=NC, num_subcores=NS)


def _zero16():
    return jnp.zeros((L,), jnp.float32)


def _zero_buf(buf, nrows):
    z = _zero16()

    def zr(r, _):
        for d in range(D // L):
            buf[r, pl.ds(d * L, L)] = z
        return 0

    lax.fori_loop(0, nrows, zr, 0)


def _edge_windows(tidx, cr_hbm, va_hbm, src, acc_sp, dummy_hbm, bufs,
                  wsz=W, nwin=NWIN):
    cr = bufs[0:3]
    va = bufs[3:6]
    rb = bufs[6:9]
    g = bufs[9:12]
    gsem = bufs[12:15]
    ssem = bufs[15:18]
    isem = bufs[18:21]
    base = tidx * nwin

    def idx_start(w, q):
        pltpu.async_copy(cr_hbm.at[base + w], cr[q], isem[q])
        pltpu.async_copy(va_hbm.at[base + w], va[q], isem[q])

    def idx_wait(w, q):
        pltpu.make_async_copy(cr_hbm.at[base + w], cr[q], isem[q]).wait()
        pltpu.make_async_copy(va_hbm.at[base + w], va[q], isem[q]).wait()

    def buf_drain(sem, q):
        pltpu.make_async_copy(dummy_hbm.at[pl.ds(0, wsz)], g[q], sem).wait()

    def scale(p):
        def sgroup(gi, _):
            sl16 = pl.ds(gi * L, L)
            rb[p][sl16] = cr[p][1, sl16]
            v16 = va[p][0, sl16]
            for j in range(L):
                vv = jnp.full((L,), v16[j], jnp.float32)
                e = gi * L + j
                for d in range(D // L):
                    sl = pl.ds(d * L, L)
                    g[p][e, sl] = g[p][e, sl] * vv
            return 0

        lax.fori_loop(0, wsz // L, sgroup, 0)

    pltpu.sync_copy(cr_hbm.at[base], cr[0])
    pltpu.sync_copy(va_hbm.at[base], va[0])
    pltpu.async_copy(src.at[cr[0].at[0]], g[0], gsem[0])
    idx_start(1, 1)
    idx_start(2, 2)

    def k_iter(k, _):
        for p in range(3):
            w = 3 * k + p
            pn = (p + 1) % 3

            @pl.when(w < nwin)
            def _():
                buf_drain(gsem[p], p)

                @pl.when(w + 1 < nwin)
                def _():
                    @pl.when(w >= 2)
                    def _():
                        buf_drain(ssem[pn], pn)
                    idx_wait(w + 1, pn)
                    pltpu.async_copy(src.at[cr[pn].at[0]], g[pn], gsem[pn])

                scale(p)
                pltpu.async_copy(g[p], acc_sp.at[rb[p]], ssem[p], add=True)

                @pl.when(w + 3 < nwin)
                def _():
                    idx_start(w + 3, p)

        return 0

    lax.fori_loop(0, (nwin + 2) // 3, k_iter, 0)
    for q in range(3):
        buf_drain(ssem[q], q)


def _phase_a_body(tag_hbm, u_cr, u_va, i_cr, i_va,
                  ui_out, tag_sp, acc_sp, *bufs):
    c = lax.axis_index("c")
    s = lax.axis_index("s")
    g0, g1 = bufs[9], bufs[10]

    for j in range((NCH_T + NS - 1) // NS):
        k = s + j * NS

        @pl.when(k < NCH_T)
        def _():
            pltpu.sync_copy(tag_hbm.at[pl.ds(k * CH, CH)], g1)
            pltpu.sync_copy(g1, tag_sp.at[pl.ds(k * CH, CH)])

    _zero_buf(g0, W)
    for j in range((NCH_A + NS - 1) // NS):
        k = s + j * NS

        @pl.when(k < NCH_A)
        def _():
            pltpu.sync_copy(g0, acc_sp.at[pl.ds(k * CH, CH)])

    plsc.subcore_barrier()

    @pl.when(c == 0)
    def _():
        _edge_windows(s, u_cr, u_va, tag_sp, acc_sp, ui_out, bufs)

    @pl.when(c == 1)
    def _():
        _edge_windows(s, i_cr, i_va, tag_sp, acc_sp, ui_out, bufs)

    plsc.subcore_barrier()
    for j in range((NCH_A + NS - 1) // NS):
        k = s + j * NS

        @pl.when(k < NCH_A)
        def _():
            pltpu.sync_copy(acc_sp.at[pl.ds(k * CH, CH)],
                            ui_out.at[pl.ds(c * NU + k * CH, CH)])


def _make_pipe_scratch(wsz):
    return (
        [pltpu.VMEM((2, wsz), jnp.int32)] * 3
        + [pltpu.VMEM((1, wsz), jnp.float32)] * 3
        + [pltpu.VMEM((wsz,), jnp.int32)] * 3
        + [pltpu.VMEM((wsz, D), jnp.float32)] * 3
        + [pltpu.SemaphoreType.DMA] * 9
    )


_pipe_scratch = _make_pipe_scratch(W)

_phase_a = pl.kernel(
    _phase_a_body,
    out_type=jax.ShapeDtypeStruct((NU + NI, D), jnp.float32),
    mesh=_mesh,
    scratch_types=[
        pltpu.VMEM_SHARED((NT, D), jnp.float32),
        pltpu.VMEM_SHARED((NU, D), jnp.float32),
    ] + _pipe_scratch,
)


def _phase_b_body(ui_raw, t_cr, t_va, tp0, tp1, acc_sp, *bufs):
    c = lax.axis_index("c")
    s = lax.axis_index("s")
    g0 = bufs[9]

    _zero_buf(g0, CH)
    for j in range((NCH_T + NS - 1) // NS):
        k = s + j * NS

        @pl.when(k < NCH_T)
        def _():
            pltpu.sync_copy(g0.at[pl.ds(0, CH)], acc_sp.at[pl.ds(k * CH, CH)])

    plsc.subcore_barrier()

    _edge_windows(c * NS + s, t_cr, t_va, ui_raw, acc_sp, ui_raw, bufs,
                  wsz=W2, nwin=NWIN2)

    plsc.subcore_barrier()

    for j in range((NCH_T + NS - 1) // NS):
        k = s + j * NS

        @pl.when((k < NCH_T) & (c == 0))
        def _():
            pltpu.sync_copy(acc_sp.at[pl.ds(k * CH, CH)],
                            tp0.at[pl.ds(k * CH, CH)])

        @pl.when((k < NCH_T) & (c == 1))
        def _():
            pltpu.sync_copy(acc_sp.at[pl.ds(k * CH, CH)],
                            tp1.at[pl.ds(k * CH, CH)])


_phase_b = pl.kernel(
    _phase_b_body,
    out_type=(jax.ShapeDtypeStruct((NT, D), jnp.float32),
              jax.ShapeDtypeStruct((NT, D), jnp.float32)),
    mesh=_mesh,
    scratch_types=[
        pltpu.VMEM_SHARED((NT, D), jnp.float32),
    ] + _make_pipe_scratch(W2),
)


def _tc_norm(x):
    n = jnp.sqrt(jnp.sum(x * x, axis=1, keepdims=True))
    return x / jnp.maximum(n, 1e-12)


def _tc_ui_body(base, x1, x2, x3, o):
    o[...] = (base[...] + _tc_norm(x1[...]) + _tc_norm(x2[...]) / 2.0
              + _tc_norm(x3[...]) / 3.0)


def _tc_tag_body(base, a0, b0, a1, b1, a2, b2, o):
    o[...] = (base[...] + _tc_norm(a0[...] + b0[...])
              + _tc_norm(a1[...] + b1[...]) / 2.0
              + _tc_norm(a2[...] + b2[...]) / 3.0)


def _tc_tagnorm_body(a, b, o):
    o[...] = _tc_norm(a[...] + b[...])


_tc_tagnorm = pl.pallas_call(
    _tc_tagnorm_body,
    out_shape=jax.ShapeDtypeStruct((NT, D), jnp.float32),
    grid=(1,),
    in_specs=[pl.BlockSpec((NT, D), lambda i: (0, 0))] * 2,
    out_specs=pl.BlockSpec((NT, D), lambda i: (0, 0)),
)


_UI_BLK = 1000
_tc_ui = pl.pallas_call(
    _tc_ui_body,
    out_shape=jax.ShapeDtypeStruct((NU + NI, D), jnp.float32),
    grid=((NU + NI) // _UI_BLK,),
    in_specs=[pl.BlockSpec((_UI_BLK, D), lambda i: (i, 0))] * 4,
    out_specs=pl.BlockSpec((_UI_BLK, D), lambda i: (i, 0)),
)

_tc_tag = pl.pallas_call(
    _tc_tag_body,
    out_shape=jax.ShapeDtypeStruct((NT, D), jnp.float32),
    grid=(1,),
    in_specs=[pl.BlockSpec((NT, D), lambda i: (0, 0))] * 7,
    out_specs=pl.BlockSpec((NT, D), lambda i: (0, 0)),
)


def kernel(user_emb, item_emb, tag_emb,
           u_rows, u_cols, u_vals,
           i_rows, i_cols, i_vals,
           t_rows, t_cols, t_vals):
    def pack(cols, rows, vals):
        cr = jnp.stack([cols.reshape(-1, W), rows.reshape(-1, W)], axis=1)
        return cr, vals.reshape(-1, 1, W)

    def pack2(cols, rows, vals):
        cr = jnp.stack([cols.reshape(-1, W2), rows.reshape(-1, W2)], axis=1)
        return cr, vals.reshape(-1, 1, W2)

    u_cr, u_va = pack(u_cols, u_rows, u_vals)
    i_cr, i_va = pack(i_cols, i_rows, i_vals)
    npad = NC * NS * EPT2 - t_cols.shape[0]
    pad = jnp.arange(npad, dtype=jnp.int32)
    t_cr, t_va = pack2(jnp.concatenate([t_cols, pad % (NU + NI)]),
                       jnp.concatenate([t_rows, pad % NT]),
                       jnp.concatenate([t_vals, jnp.zeros((npad,),
                                                          t_vals.dtype)]))

    ui1 = _phase_a(tag_emb, u_cr, u_va, i_cr, i_va)
    tp0_0, tp1_0 = _phase_b(ui1, t_cr, t_va)
    tag1 = _tc_tagnorm(tp0_0, tp1_0)
    ui2 = _phase_a(tag1, u_cr, u_va, i_cr, i_va)
    tp0_1, tp1_1 = _phase_b(ui2, t_cr, t_va)
    tag2 = _tc_tagnorm(tp0_1, tp1_1)
    ui3 = _phase_a(tag2, u_cr, u_va, i_cr, i_va)
    tp0_2, tp1_2 = _phase_b(ui3, t_cr, t_va)

    base_ui = jnp.concatenate([user_emb, item_emb], axis=0)
    out_ui = _tc_ui(base_ui, ui1, ui2, ui3)
    out_t = _tc_tag(tag_emb, tp0_0, tp1_0, tp0_1, tp1_1, tp0_2, tp1_2)
    return (out_ui[:NU], out_ui[NU:], out_t)


# speedup vs baseline: 2.5099x; 1.0020x over previous
import functools
import jax
import jax.numpy as jnp
from jax import lax
from jax.experimental import pallas as pl
from jax.experimental.pallas import tpu as pltpu
from jax.experimental.pallas import tpu_sc as plsc

NU = 10000
NI = 10000
NT = 2000
D = 128
NC = 2
NS = 16
L = 16

W = 80
EPT = 10000
NWIN = EPT // W
CH = 80
NCH_T = NT // CH
NCH_A = NU // CH
W2 = 128
NWIN2 = 80
EPT2 = W2 * NWIN2

_mesh = plsc.VectorSubcoreMesh(
    core_axis_name="c", subcore_axis_name="s", num_cores=NC, num_subcores=NS)


def _zero16():
    return jnp.zeros((L,), jnp.float32)


def _zero_buf(buf, nrows):
    z = _zero16()

    def zr(r, _):
        for d in range(D // L):
            buf[r, pl.ds(d * L, L)] = z
        return 0

    lax.fori_loop(0, nrows, zr, 0)


def _edge_windows(tidx, cr_hbm, va_hbm, src, acc_sp, dummy_hbm, bufs,
                  wsz=W, nwin=NWIN):
    cr = bufs[0:3]
    va = bufs[3:6]
    rb = bufs[6:9]
    g = bufs[9:12]
    gsem = bufs[12:15]
    ssem = bufs[15:18]
    isem = bufs[18:21]
    base = tidx * nwin

    def idx_start(w, q):
        pltpu.async_copy(cr_hbm.at[base + w], cr[q], isem[q])
        pltpu.async_copy(va_hbm.at[base + w], va[q], isem[q])

    def idx_wait(w, q):
        pltpu.make_async_copy(cr_hbm.at[base + w], cr[q], isem[q]).wait()
        pltpu.make_async_copy(va_hbm.at[base + w], va[q], isem[q]).wait()

    def buf_drain(sem, q):
        pltpu.make_async_copy(dummy_hbm.at[pl.ds(0, wsz)], g[q], sem).wait()

    def scale(p):
        def sgroup(gi, _):
            sl16 = pl.ds(gi * L, L)
            rb[p][sl16] = cr[p][1, sl16]
            v16 = va[p][0, sl16]
            for j in range(L):
                vv = jnp.full((L,), v16[j], jnp.float32)
                e = gi * L + j
                for d in range(D // L):
                    sl = pl.ds(d * L, L)
                    g[p][e, sl] = g[p][e, sl] * vv
            return 0

        lax.fori_loop(0, wsz // L, sgroup, 0)

    pltpu.sync_copy(cr_hbm.at[base], cr[0])
    pltpu.sync_copy(va_hbm.at[base], va[0])
    pltpu.async_copy(src.at[cr[0].at[0]], g[0], gsem[0])
    idx_start(1, 1)
    idx_start(2, 2)

    def k_iter(k, _):
        for p in range(3):
            w = 3 * k + p
            pn = (p + 1) % 3

            @pl.when(w < nwin)
            def _():
                buf_drain(gsem[p], p)

                @pl.when(w + 1 < nwin)
                def _():
                    @pl.when(w >= 2)
                    def _():
                        buf_drain(ssem[pn], pn)
                    idx_wait(w + 1, pn)
                    pltpu.async_copy(src.at[cr[pn].at[0]], g[pn], gsem[pn])

                scale(p)
                pltpu.async_copy(g[p], acc_sp.at[rb[p]], ssem[p], add=True)

                @pl.when(w + 3 < nwin)
                def _():
                    idx_start(w + 3, p)

        return 0

    lax.fori_loop(0, (nwin + 2) // 3, k_iter, 0)
    for q in range(3):
        buf_drain(ssem[q], q)


def _phase_a_body(tag_hbm, u_cr, u_va, i_cr, i_va,
                  ui_out, tag_sp, acc_sp, *bufs):
    c = lax.axis_index("c")
    s = lax.axis_index("s")
    g0, g1, g2 = bufs[9], bufs[10], bufs[11]
    psem = bufs[18]

    stage_bufs = (g1, g2)
    for j in range((NCH_T + NS - 1) // NS):
        k = s + j * NS

        @pl.when(k < NCH_T)
        def _():
            pltpu.sync_copy(tag_hbm.at[pl.ds(k * CH, CH)], stage_bufs[j])
            pltpu.async_copy(stage_bufs[j], tag_sp.at[pl.ds(k * CH, CH)],
                             psem)

    _zero_buf(g0, W)
    for j in range((NCH_A + NS - 1) // NS):
        k = s + j * NS

        @pl.when(k < NCH_A)
        def _():
            pltpu.async_copy(g0, acc_sp.at[pl.ds(k * CH, CH)], psem)

    for j in range((NCH_T + NS - 1) // NS):
        k = s + j * NS

        @pl.when(k < NCH_T)
        def _():
            pltpu.make_async_copy(stage_bufs[j],
                                  tag_sp.at[pl.ds(k * CH, CH)], psem).wait()
    for j in range((NCH_A + NS - 1) // NS):
        k = s + j * NS

        @pl.when(k < NCH_A)
        def _():
            pltpu.make_async_copy(g0, acc_sp.at[pl.ds(k * CH, CH)],
                                  psem).wait()

    plsc.subcore_barrier()

    @pl.when(c == 0)
    def _():
        _edge_windows(s, u_cr, u_va, tag_sp, acc_sp, ui_out, bufs)

    @pl.when(c == 1)
    def _():
        _edge_windows(s, i_cr, i_va, tag_sp, acc_sp, ui_out, bufs)

    plsc.subcore_barrier()
    for j in range((NCH_A + NS - 1) // NS):
        k = s + j * NS

        @pl.when(k < NCH_A)
        def _():
            pltpu.async_copy(acc_sp.at[pl.ds(k * CH, CH)],
                             ui_out.at[pl.ds(c * NU + k * CH, CH)], psem)
    for j in range((NCH_A + NS - 1) // NS):
        k = s + j * NS

        @pl.when(k < NCH_A)
        def _():
            pltpu.make_async_copy(acc_sp.at[pl.ds(k * CH, CH)],
                                  ui_out.at[pl.ds(c * NU + k * CH, CH)],
                                  psem).wait()


def _make_pipe_scratch(wsz):
    return (
        [pltpu.VMEM((2, wsz), jnp.int32)] * 3
        + [pltpu.VMEM((1, wsz), jnp.float32)] * 3
        + [pltpu.VMEM((wsz,), jnp.int32)] * 3
        + [pltpu.VMEM((wsz, D), jnp.float32)] * 3
        + [pltpu.SemaphoreType.DMA] * 9
    )


_pipe_scratch = _make_pipe_scratch(W)

_phase_a = pl.kernel(
    _phase_a_body,
    out_type=jax.ShapeDtypeStruct((NU + NI, D), jnp.float32),
    mesh=_mesh,
    scratch_types=[
        pltpu.VMEM_SHARED((NT, D), jnp.float32),
        pltpu.VMEM_SHARED((NU, D), jnp.float32),
    ] + _pipe_scratch,
)


def _phase_b_body(ui_raw, t_cr, t_va, tp0, tp1, acc_sp, *bufs):
    c = lax.axis_index("c")
    s = lax.axis_index("s")
    g0 = bufs[9]

    psem = bufs[18]
    _zero_buf(g0, CH)
    for j in range((NCH_T + NS - 1) // NS):
        k = s + j * NS

        @pl.when(k < NCH_T)
        def _():
            pltpu.async_copy(g0.at[pl.ds(0, CH)],
                             acc_sp.at[pl.ds(k * CH, CH)], psem)
    for j in range((NCH_T + NS - 1) // NS):
        k = s + j * NS

        @pl.when(k < NCH_T)
        def _():
            pltpu.make_async_copy(g0.at[pl.ds(0, CH)],
                                  acc_sp.at[pl.ds(k * CH, CH)], psem).wait()

    plsc.subcore_barrier()

    _edge_windows(c * NS + s, t_cr, t_va, ui_raw, acc_sp, ui_raw, bufs,
                  wsz=W2, nwin=NWIN2)

    plsc.subcore_barrier()

    for j in range((NCH_T + NS - 1) // NS):
        k = s + j * NS

        @pl.when((k < NCH_T) & (c == 0))
        def _():
            pltpu.sync_copy(acc_sp.at[pl.ds(k * CH, CH)],
                            tp0.at[pl.ds(k * CH, CH)])

        @pl.when((k < NCH_T) & (c == 1))
        def _():
            pltpu.sync_copy(acc_sp.at[pl.ds(k * CH, CH)],
                            tp1.at[pl.ds(k * CH, CH)])


_phase_b = pl.kernel(
    _phase_b_body,
    out_type=(jax.ShapeDtypeStruct((NT, D), jnp.float32),
              jax.ShapeDtypeStruct((NT, D), jnp.float32)),
    mesh=_mesh,
    scratch_types=[
        pltpu.VMEM_SHARED((NT, D), jnp.float32),
    ] + _make_pipe_scratch(W2),
)


def _tc_norm(x):
    n = jnp.sqrt(jnp.sum(x * x, axis=1, keepdims=True))
    return x / jnp.maximum(n, 1e-12)


def _tc_ui_body(base, x1, x2, x3, o):
    o[...] = (base[...] + _tc_norm(x1[...]) + _tc_norm(x2[...]) / 2.0
              + _tc_norm(x3[...]) / 3.0)


def _tc_tag_body(base, a0, b0, a1, b1, a2, b2, o):
    o[...] = (base[...] + _tc_norm(a0[...] + b0[...])
              + _tc_norm(a1[...] + b1[...]) / 2.0
              + _tc_norm(a2[...] + b2[...]) / 3.0)


def _tc_tagnorm_body(a, b, o):
    o[...] = _tc_norm(a[...] + b[...])


_tc_tagnorm = pl.pallas_call(
    _tc_tagnorm_body,
    out_shape=jax.ShapeDtypeStruct((NT, D), jnp.float32),
    grid=(1,),
    in_specs=[pl.BlockSpec((NT, D), lambda i: (0, 0))] * 2,
    out_specs=pl.BlockSpec((NT, D), lambda i: (0, 0)),
)


_UI_BLK = 1000
_tc_ui = pl.pallas_call(
    _tc_ui_body,
    out_shape=jax.ShapeDtypeStruct((NU + NI, D), jnp.float32),
    grid=((NU + NI) // _UI_BLK,),
    in_specs=[pl.BlockSpec((_UI_BLK, D), lambda i: (i, 0))] * 4,
    out_specs=pl.BlockSpec((_UI_BLK, D), lambda i: (i, 0)),
)

_tc_tag = pl.pallas_call(
    _tc_tag_body,
    out_shape=jax.ShapeDtypeStruct((NT, D), jnp.float32),
    grid=(1,),
    in_specs=[pl.BlockSpec((NT, D), lambda i: (0, 0))] * 7,
    out_specs=pl.BlockSpec((NT, D), lambda i: (0, 0)),
)


def kernel(user_emb, item_emb, tag_emb,
           u_rows, u_cols, u_vals,
           i_rows, i_cols, i_vals,
           t_rows, t_cols, t_vals):
    def pack(cols, rows, vals):
        cr = jnp.stack([cols.reshape(-1, W), rows.reshape(-1, W)], axis=1)
        return cr, vals.reshape(-1, 1, W)

    def pack2(cols, rows, vals):
        cr = jnp.stack([cols.reshape(-1, W2), rows.reshape(-1, W2)], axis=1)
        return cr, vals.reshape(-1, 1, W2)

    u_cr, u_va = pack(u_cols, u_rows, u_vals)
    i_cr, i_va = pack(i_cols, i_rows, i_vals)
    npad = NC * NS * EPT2 - t_cols.shape[0]
    pad = jnp.arange(npad, dtype=jnp.int32)
    t_cr, t_va = pack2(jnp.concatenate([t_cols, pad % (NU + NI)]),
                       jnp.concatenate([t_rows, pad % NT]),
                       jnp.concatenate([t_vals, jnp.zeros((npad,),
                                                          t_vals.dtype)]))

    ui1 = _phase_a(tag_emb, u_cr, u_va, i_cr, i_va)
    tp0_0, tp1_0 = _phase_b(ui1, t_cr, t_va)
    tag1 = _tc_tagnorm(tp0_0, tp1_0)
    ui2 = _phase_a(tag1, u_cr, u_va, i_cr, i_va)
    tp0_1, tp1_1 = _phase_b(ui2, t_cr, t_va)
    tag2 = _tc_tagnorm(tp0_1, tp1_1)
    ui3 = _phase_a(tag2, u_cr, u_va, i_cr, i_va)
    tp0_2, tp1_2 = _phase_b(ui3, t_cr, t_va)

    base_ui = jnp.concatenate([user_emb, item_emb], axis=0)
    out_ui = _tc_ui(base_ui, ui1, ui2, ui3)
    out_t = _tc_tag(tag_emb, tp0_0, tp1_0, tp0_1, tp1_1, tp0_2, tp1_2)
    return (out_ui[:NU], out_ui[NU:], out_t)


# speedup vs baseline: 2.6409x; 1.0522x over previous
import functools
import jax
import jax.numpy as jnp
from jax import lax
from jax.experimental import pallas as pl
from jax.experimental.pallas import tpu as pltpu
from jax.experimental.pallas import tpu_sc as plsc

NU = 10000
NI = 10000
NT = 2000
D = 128
NC = 2
NS = 16
L = 16

W = 128
NWIN = 80
EPT = W * NWIN
CH = 80
NCH_T = NT // CH
NCH_A = NU // CH

_mesh = plsc.VectorSubcoreMesh(
    core_axis_name="c", subcore_axis_name="s", num_cores=NC, num_subcores=NS)


def _zero16():
    return jnp.zeros((L,), jnp.float32)


def _zero_buf(buf, nrows):
    z = _zero16()

    def zr(r, _):
        for d in range(D // L):
            buf[r, pl.ds(d * L, L)] = z
        return 0

    lax.fori_loop(0, nrows, zr, 0)


def _edge_windows(tidx, cr_hbm, va_hbm, src, acc_sp, dummy_hbm, bufs,
                  wsz=W, nwin=NWIN):
    cr = bufs[0:3]
    va = bufs[3:6]
    rb = bufs[6:9]
    g = bufs[9:12]
    gsem = bufs[12:15]
    ssem = bufs[15:18]
    isem = bufs[18:21]
    base = tidx * nwin

    def idx_start(w, q):
        pltpu.async_copy(cr_hbm.at[base + w], cr[q], isem[q])
        pltpu.async_copy(va_hbm.at[base + w], va[q], isem[q])

    def idx_wait(w, q):
        pltpu.make_async_copy(cr_hbm.at[base + w], cr[q], isem[q]).wait()
        pltpu.make_async_copy(va_hbm.at[base + w], va[q], isem[q]).wait()

    def buf_drain(sem, q):
        pltpu.make_async_copy(dummy_hbm.at[pl.ds(0, wsz)], g[q], sem).wait()

    def scale(p):
        def sgroup(gi, _):
            sl16 = pl.ds(gi * L, L)
            rb[p][sl16] = cr[p][1, sl16]
            v16 = va[p][0, sl16]
            for j in range(L):
                vv = jnp.full((L,), v16[j], jnp.float32)
                e = gi * L + j
                for d in range(D // L):
                    sl = pl.ds(d * L, L)
                    g[p][e, sl] = g[p][e, sl] * vv
            return 0

        lax.fori_loop(0, wsz // L, sgroup, 0)

    pltpu.sync_copy(cr_hbm.at[base], cr[0])
    pltpu.sync_copy(va_hbm.at[base], va[0])
    pltpu.async_copy(src.at[cr[0].at[0]], g[0], gsem[0])
    idx_start(1, 1)
    idx_start(2, 2)

    def k_iter(k, _):
        for p in range(3):
            w = 3 * k + p
            pn = (p + 1) % 3

            @pl.when(w < nwin)
            def _():
                buf_drain(gsem[p], p)

                @pl.when(w + 1 < nwin)
                def _():
                    @pl.when(w >= 2)
                    def _():
                        buf_drain(ssem[pn], pn)
                    idx_wait(w + 1, pn)
                    pltpu.async_copy(src.at[cr[pn].at[0]], g[pn], gsem[pn])

                scale(p)
                pltpu.async_copy(g[p], acc_sp.at[rb[p]], ssem[p], add=True)

                @pl.when(w + 3 < nwin)
                def _():
                    idx_start(w + 3, p)

        return 0

    lax.fori_loop(0, (nwin + 2) // 3, k_iter, 0)
    for q in range(3):
        buf_drain(ssem[q], q)


def _phase_a_body(tag_hbm, u_cr, u_va, i_cr, i_va,
                  ui_out, acc_sp, *bufs):
    c = lax.axis_index("c")
    s = lax.axis_index("s")
    g0 = bufs[9]
    psem = bufs[18]

    _zero_buf(g0, CH)
    for j in range((NCH_A + NS - 1) // NS):
        k = s + j * NS

        @pl.when(k < NCH_A)
        def _():
            pltpu.async_copy(g0.at[pl.ds(0, CH)],
                             acc_sp.at[pl.ds(k * CH, CH)], psem)
    for j in range((NCH_A + NS - 1) // NS):
        k = s + j * NS

        @pl.when(k < NCH_A)
        def _():
            pltpu.make_async_copy(g0.at[pl.ds(0, CH)],
                                  acc_sp.at[pl.ds(k * CH, CH)],
                                  psem).wait()

    plsc.subcore_barrier()

    @pl.when(c == 0)
    def _():
        _edge_windows(s, u_cr, u_va, tag_hbm, acc_sp, ui_out, bufs)

    @pl.when(c == 1)
    def _():
        _edge_windows(s, i_cr, i_va, tag_hbm, acc_sp, ui_out, bufs)

    plsc.subcore_barrier()
    for j in range((NCH_A + NS - 1) // NS):
        k = s + j * NS

        @pl.when(k < NCH_A)
        def _():
            pltpu.async_copy(acc_sp.at[pl.ds(k * CH, CH)],
                             ui_out.at[pl.ds(c * NU + k * CH, CH)], psem)
    for j in range((NCH_A + NS - 1) // NS):
        k = s + j * NS

        @pl.when(k < NCH_A)
        def _():
            pltpu.make_async_copy(acc_sp.at[pl.ds(k * CH, CH)],
                                  ui_out.at[pl.ds(c * NU + k * CH, CH)],
                                  psem).wait()


def _make_pipe_scratch(wsz):
    return (
        [pltpu.VMEM((2, wsz), jnp.int32)] * 3
        + [pltpu.VMEM((1, wsz), jnp.float32)] * 3
        + [pltpu.VMEM((wsz,), jnp.int32)] * 3
        + [pltpu.VMEM((wsz, D), jnp.float32)] * 3
        + [pltpu.SemaphoreType.DMA] * 9
    )


_pipe_scratch = _make_pipe_scratch(W)

_phase_a = pl.kernel(
    _phase_a_body,
    out_type=jax.ShapeDtypeStruct((NU + NI, D), jnp.float32),
    mesh=_mesh,
    scratch_types=[
        pltpu.VMEM_SHARED((NU, D), jnp.float32),
    ] + _pipe_scratch,
)


def _phase_b_body(ui_raw, t_cr, t_va, tp0, tp1, acc_sp, *bufs):
    c = lax.axis_index("c")
    s = lax.axis_index("s")
    g0 = bufs[9]

    psem = bufs[18]
    _zero_buf(g0, CH)
    for j in range((NCH_T + NS - 1) // NS):
        k = s + j * NS

        @pl.when(k < NCH_T)
        def _():
            pltpu.async_copy(g0.at[pl.ds(0, CH)],
                             acc_sp.at[pl.ds(k * CH, CH)], psem)
    for j in range((NCH_T + NS - 1) // NS):
        k = s + j * NS

        @pl.when(k < NCH_T)
        def _():
            pltpu.make_async_copy(g0.at[pl.ds(0, CH)],
                                  acc_sp.at[pl.ds(k * CH, CH)], psem).wait()

    plsc.subcore_barrier()

    _edge_windows(c * NS + s, t_cr, t_va, ui_raw, acc_sp, ui_raw, bufs)

    plsc.subcore_barrier()

    for j in range((NCH_T + NS - 1) // NS):
        k = s + j * NS

        @pl.when((k < NCH_T) & (c == 0))
        def _():
            pltpu.sync_copy(acc_sp.at[pl.ds(k * CH, CH)],
                            tp0.at[pl.ds(k * CH, CH)])

        @pl.when((k < NCH_T) & (c == 1))
        def _():
            pltpu.sync_copy(acc_sp.at[pl.ds(k * CH, CH)],
                            tp1.at[pl.ds(k * CH, CH)])


_phase_b = pl.kernel(
    _phase_b_body,
    out_type=(jax.ShapeDtypeStruct((NT, D), jnp.float32),
              jax.ShapeDtypeStruct((NT, D), jnp.float32)),
    mesh=_mesh,
    scratch_types=[
        pltpu.VMEM_SHARED((NT, D), jnp.float32),
    ] + _make_pipe_scratch(W),
)


def _tc_norm(x):
    n = jnp.sqrt(jnp.sum(x * x, axis=1, keepdims=True))
    return x / jnp.maximum(n, 1e-12)


def _tc_ui_body(base, x1, x2, x3, o):
    o[...] = (base[...] + _tc_norm(x1[...]) + _tc_norm(x2[...]) / 2.0
              + _tc_norm(x3[...]) / 3.0)


def _tc_tag_body(base, a0, b0, a1, b1, a2, b2, o):
    o[...] = (base[...] + _tc_norm(a0[...] + b0[...])
              + _tc_norm(a1[...] + b1[...]) / 2.0
              + _tc_norm(a2[...] + b2[...]) / 3.0)


def _tc_tagnorm_body(a, b, o):
    o[...] = _tc_norm(a[...] + b[...])


_tc_tagnorm = pl.pallas_call(
    _tc_tagnorm_body,
    out_shape=jax.ShapeDtypeStruct((NT, D), jnp.float32),
    grid=(1,),
    in_specs=[pl.BlockSpec((NT, D), lambda i: (0, 0))] * 2,
    out_specs=pl.BlockSpec((NT, D), lambda i: (0, 0)),
)


_UI_BLK = 1000
_tc_ui = pl.pallas_call(
    _tc_ui_body,
    out_shape=jax.ShapeDtypeStruct((NU + NI, D), jnp.float32),
    grid=((NU + NI) // _UI_BLK,),
    in_specs=[pl.BlockSpec((_UI_BLK, D), lambda i: (i, 0))] * 4,
    out_specs=pl.BlockSpec((_UI_BLK, D), lambda i: (i, 0)),
)

_tc_tag = pl.pallas_call(
    _tc_tag_body,
    out_shape=jax.ShapeDtypeStruct((NT, D), jnp.float32),
    grid=(1,),
    in_specs=[pl.BlockSpec((NT, D), lambda i: (0, 0))] * 7,
    out_specs=pl.BlockSpec((NT, D), lambda i: (0, 0)),
)


def kernel(user_emb, item_emb, tag_emb,
           u_rows, u_cols, u_vals,
           i_rows, i_cols, i_vals,
           t_rows, t_cols, t_vals):
    def pack(cols, rows, vals, nworkers, nsrc, ndst):
        npad = nworkers * EPT - cols.shape[0]
        pad = jnp.arange(npad, dtype=jnp.int32)
        cols = jnp.concatenate([cols, pad % nsrc])
        rows = jnp.concatenate([rows, pad % ndst])
        vals = jnp.concatenate([vals, jnp.zeros((npad,), vals.dtype)])
        cr = jnp.stack([cols.reshape(-1, W), rows.reshape(-1, W)], axis=1)
        return cr, vals.reshape(-1, 1, W)

    u_cr, u_va = pack(u_cols, u_rows, u_vals, NS, NT, NU)
    i_cr, i_va = pack(i_cols, i_rows, i_vals, NS, NT, NI)
    t_cr, t_va = pack(t_cols, t_rows, t_vals, NC * NS, NU + NI, NT)

    ui1 = _phase_a(tag_emb, u_cr, u_va, i_cr, i_va)
    tp0_0, tp1_0 = _phase_b(ui1, t_cr, t_va)
    tag1 = _tc_tagnorm(tp0_0, tp1_0)
    ui2 = _phase_a(tag1, u_cr, u_va, i_cr, i_va)
    tp0_1, tp1_1 = _phase_b(ui2, t_cr, t_va)
    tag2 = _tc_tagnorm(tp0_1, tp1_1)
    ui3 = _phase_a(tag2, u_cr, u_va, i_cr, i_va)
    tp0_2, tp1_2 = _phase_b(ui3, t_cr, t_va)

    base_ui = jnp.concatenate([user_emb, item_emb], axis=0)
    out_ui = _tc_ui(base_ui, ui1, ui2, ui3)
    out_t = _tc_tag(tag_emb, tp0_0, tp1_0, tp0_1, tp1_1, tp0_2, tp1_2)
    return (out_ui[:NU], out_ui[NU:], out_t)
